# baseline (speedup 1.0000x reference)
import functools

import numpy as np
import jax
import jax.numpy as jnp
from jax import lax
from jax.experimental import pallas as pl
from jax.experimental.pallas import tpu as pltpu
from jax.experimental.pallas import tpu_sc as plsc

_CRITERIA_TO_HEAD = [
    [0, 1, 2, 3, 4, 5, 6, 7, 0, 1, 2, 3],
    [1, 2, 3, 4, 5, 6, 7, 0],
    [2, 3, 4, 5, 6, 7, 0, 1, 2, 3, 4, 5, 6, 7, 0, 1],
    [3, 4, 5, 6],
    [4, 5, 6, 7, 0, 1, 2, 3, 4, 5],
    [5, 6, 7, 0, 1, 2],
    [6, 7, 0, 1, 2, 3, 4, 5, 6, 7, 0, 1, 2, 3],
    [7, 0],
]
_L = 16
_P = len(_CRITERIA_TO_HEAD)
_H = 8

_LEN_NP = np.array([len(m) for m in _CRITERIA_TO_HEAD], dtype=np.int32)
_HM_NP = np.zeros((_P, _L), dtype=np.int32)
for _i, _m in enumerate(_CRITERIA_TO_HEAD):
    _HM_NP[_i, : len(_m)] = np.array(_m, dtype=np.int32)
_ONEHOT_NP = (_HM_NP.reshape(-1, 1) == np.arange(_H)[None, :]).astype(np.float32)
_VALID_NP = (np.arange(_L)[None, :] < _LEN_NP[:, None]).astype(np.float32)
_NEGINF_NP = ((1.0 - _VALID_NP) * -1e30).astype(np.float32).reshape(-1)

_NC, _NS, _LANES = 2, 16, 16
_NW = _NC * _NS


def _tc_body(x_ref, pone_ref, cf_ref, b_ref, wt_ref, oh_ref, ninf_ref,
             hm_ref, valid_ref, e_ref, hmc2_ref, mask_ref, c2tab_v):
    wt = wt_ref[...]
    e_ref[...] = jnp.dot(x_ref[...], wt, preferred_element_type=jnp.float32)

    @pl.when(pl.program_id(0) == 0)
    def _():
        cw = jnp.dot(cf_ref[...], wt, preferred_element_type=jnp.float32)
        onehot = oh_ref[...]
        bsel = jnp.sum(onehot * b_ref[...], axis=1)
        c2 = jnp.sum(cw * onehot, axis=1) + bsel + ninf_ref[0, :]
        c2tab_v[...] = c2.reshape(_P, _L)

    pone = pone_ref[...]
    hmr = jnp.dot(pone, hm_ref[...], preferred_element_type=jnp.float32)
    c2r = jnp.dot(pone, c2tab_v[...], preferred_element_type=jnp.float32)
    hmc2_ref[...] = jnp.concatenate([hmr, c2r], axis=1)
    mask_ref[...] = jnp.dot(pone, valid_ref[...],
                            preferred_element_type=jnp.float32)


def _tc_stage(emb, pone, crit_flat, b_row, wt):
    rows, d = emb.shape
    h = wt.shape[1]
    blk = 1024
    grid = rows // blk
    return pl.pallas_call(
        _tc_body,
        grid=(grid,),
        in_specs=[
            pl.BlockSpec((blk, d), lambda i: (i, 0)),
            pl.BlockSpec((blk, _P), lambda i: (i, 0)),
            pl.BlockSpec((_P * _L, d), lambda i: (0, 0)),
            pl.BlockSpec((1, _H), lambda i: (0, 0)),
            pl.BlockSpec((d, h), lambda i: (0, 0)),
            pl.BlockSpec((_P * _L, _H), lambda i: (0, 0)),
            pl.BlockSpec((1, _P * _L), lambda i: (0, 0)),
            pl.BlockSpec((_P, _L), lambda i: (0, 0)),
            pl.BlockSpec((_P, _L), lambda i: (0, 0)),
        ],
        out_specs=[
            pl.BlockSpec((blk, h), lambda i: (i, 0)),
            pl.BlockSpec((blk, 2 * _L), lambda i: (i, 0)),
            pl.BlockSpec((blk, _L), lambda i: (i, 0)),
        ],
        out_shape=[
            jax.ShapeDtypeStruct((rows, h), jnp.float32),
            jax.ShapeDtypeStruct((rows, 2 * _L), jnp.float32),
            jax.ShapeDtypeStruct((rows, _L), jnp.float32),
        ],
        scratch_shapes=[pltpu.VMEM((_P, _L), jnp.float32)],
    )(emb, pone, crit_flat, b_row, wt,
      jnp.asarray(_ONEHOT_NP), jnp.asarray(_NEGINF_NP).reshape(1, _P * _L),
      jnp.asarray(_HM_NP.astype(np.float32)), jnp.asarray(_VALID_NP))


def _make_sc_route(batch):
    ipw = batch // _NW
    mesh = plsc.VectorSubcoreMesh(
        core_axis_name="c", subcore_axis_name="s",
        num_cores=_NC, num_subcores=_NS,
    )

    @functools.partial(
        pl.kernel,
        mesh=mesh,
        compiler_params=pltpu.CompilerParams(
            needs_layout_passes=False,
            skip_device_barrier=True,
            disable_bounds_checks=True,
        ),
        out_type=(jax.ShapeDtypeStruct((batch * _L,), jnp.float32),),
        scratch_types=(
            pltpu.VMEM((ipw * _H,), jnp.float32),
            pltpu.VMEM((ipw * 2 * _L,), jnp.float32),
            pltpu.VMEM((ipw * _L,), jnp.float32),
            pltpu.SemaphoreType.DMA,
            pltpu.SemaphoreType.DMA,
        ),
    )
    def route(e_hbm, hmc2_hbm, out_hbm, e_v, t_v, out_v, s0, s1):
        wid = lax.axis_index("s") * _NC + lax.axis_index("c")
        base = wid * ipw
        d0 = pltpu.async_copy(e_hbm.at[pl.ds(base * _H, ipw * _H)], e_v, s0)
        d1 = pltpu.async_copy(
            hmc2_hbm.at[pl.ds(base * 2 * _L, ipw * 2 * _L)], t_v, s1)
        d0.wait()
        d1.wait()

        @plsc.parallel_loop(0, ipw, 1, unroll=8)
        def item(i):
            hm_row = t_v[pl.ds(i * 2 * _L, _L)].astype(jnp.int32)
            c2row = t_v[pl.ds(i * 2 * _L + _L, _L)]
            ivec = jnp.full((_LANES,), i, jnp.int32)
            e_row = plsc.load_gather(e_v, [ivec * _H + hm_row])
            out_v[pl.ds(i * _L, _L)] = 1.0 / (1.0 + jnp.exp(-(e_row + c2row)))

        o0 = pltpu.async_copy(out_v, out_hbm.at[pl.ds(base * _L, ipw * _L)], s0)
        o0.wait()

    return route


def kernel(embedding, criteria, W, b, problem_indices):
    batch, d = embedding.shape
    crit_flat = criteria.reshape(_P * _L, d)
    pone = (problem_indices[:, None] == jnp.arange(_P, dtype=jnp.int32)
            ).astype(jnp.float32)
    e_scores, hmc2, mask = _tc_stage(
        embedding, pone, crit_flat, b.reshape(1, _H).astype(jnp.float32), W.T)

    route = _make_sc_route(batch)
    (out_flat,) = route(e_scores.reshape(-1), hmc2.reshape(-1))
    return out_flat.reshape(batch, _L), mask


# speedup vs baseline: 1.1205x; 1.1205x over previous
import functools

import numpy as np
import jax
import jax.numpy as jnp
from jax import lax
from jax.experimental import pallas as pl
from jax.experimental.pallas import tpu as pltpu
from jax.experimental.pallas import tpu_sc as plsc

_CRITERIA_TO_HEAD = [
    [0, 1, 2, 3, 4, 5, 6, 7, 0, 1, 2, 3],
    [1, 2, 3, 4, 5, 6, 7, 0],
    [2, 3, 4, 5, 6, 7, 0, 1, 2, 3, 4, 5, 6, 7, 0, 1],
    [3, 4, 5, 6],
    [4, 5, 6, 7, 0, 1, 2, 3, 4, 5],
    [5, 6, 7, 0, 1, 2],
    [6, 7, 0, 1, 2, 3, 4, 5, 6, 7, 0, 1, 2, 3],
    [7, 0],
]
_L = 16
_P = len(_CRITERIA_TO_HEAD)
_H = 8

_LEN_NP = np.array([len(m) for m in _CRITERIA_TO_HEAD], dtype=np.int32)
_HM_NP = np.zeros((_P, _L), dtype=np.int32)
for _i, _m in enumerate(_CRITERIA_TO_HEAD):
    _HM_NP[_i, : len(_m)] = np.array(_m, dtype=np.int32)
_ONEHOT_NP = (_HM_NP.reshape(-1, 1) == np.arange(_H)[None, :]).astype(np.float32)
_NEGINF_NP = np.where(
    np.arange(_L)[None, :] < _LEN_NP[:, None], 0.0, -1e30
).astype(np.float32).reshape(-1)

_NC, _NS, _LANES = 2, 16, 16
_NW = _NC * _NS


def _mm_body(x_ref, cf_ref, b_ref, wt_ref, oh_ref, ninf_ref, hm_ref,
             e_ref, c2_ref):
    wt = wt_ref[...]
    e_ref[...] = jnp.dot(x_ref[...], wt, preferred_element_type=jnp.float32)

    @pl.when(pl.program_id(0) == 0)
    def _():
        cw = jnp.dot(cf_ref[...], wt, preferred_element_type=jnp.float32)
        onehot = oh_ref[...]
        bsel = jnp.sum(onehot * b_ref[...], axis=1)
        c2 = jnp.sum(cw * onehot, axis=1) + bsel + ninf_ref[0, :]
        ci = lax.bitcast_convert_type(c2.reshape(1, _P * _L), jnp.int32)
        c2_ref[...] = (ci & jnp.int32(-8)) | hm_ref[...]


def _scores_matmul(emb, crit_flat, b_row, wt):
    rows, d = emb.shape
    h = wt.shape[1]
    blk = 1024
    grid = rows // blk
    return pl.pallas_call(
        _mm_body,
        grid=(grid,),
        in_specs=[
            pl.BlockSpec((blk, d), lambda i: (i, 0)),
            pl.BlockSpec((_P * _L, d), lambda i: (0, 0)),
            pl.BlockSpec((1, _H), lambda i: (0, 0)),
            pl.BlockSpec((d, h), lambda i: (0, 0)),
            pl.BlockSpec((_P * _L, _H), lambda i: (0, 0)),
            pl.BlockSpec((1, _P * _L), lambda i: (0, 0)),
            pl.BlockSpec((1, _P * _L), lambda i: (0, 0)),
        ],
        out_specs=[
            pl.BlockSpec((blk, h), lambda i: (i, 0)),
            pl.BlockSpec((1, _P * _L), lambda i: (0, 0)),
        ],
        out_shape=[
            jax.ShapeDtypeStruct((rows, h), jnp.float32),
            jax.ShapeDtypeStruct((1, _P * _L), jnp.int32),
        ],
    )(emb, crit_flat, b_row, wt,
      jnp.asarray(_ONEHOT_NP), jnp.asarray(_NEGINF_NP).reshape(1, _P * _L),
      jnp.asarray(_HM_NP.reshape(1, _P * _L)))


def _make_sc_route(batch):
    ipw = batch // _NW
    mesh = plsc.VectorSubcoreMesh(
        core_axis_name="c", subcore_axis_name="s",
        num_cores=_NC, num_subcores=_NS,
    )

    @functools.partial(
        pl.kernel,
        mesh=mesh,
        compiler_params=pltpu.CompilerParams(
            needs_layout_passes=False,
            skip_device_barrier=True,
            disable_bounds_checks=True,
        ),
        out_type=(
            jax.ShapeDtypeStruct((batch * _L,), jnp.float32),
            jax.ShapeDtypeStruct((batch * _L,), jnp.float32),
        ),
        scratch_types=(
            pltpu.VMEM((ipw * _H,), jnp.float32),
            pltpu.VMEM((ipw,), jnp.int32),
            pltpu.VMEM((_P * _L,), jnp.int32),
            pltpu.VMEM((ipw * _L,), jnp.float32),
            pltpu.VMEM((ipw * _L,), jnp.float32),
            pltpu.SemaphoreType.DMA,
            pltpu.SemaphoreType.DMA,
            pltpu.SemaphoreType.DMA,
        ),
    )
    def route(e_hbm, p_hbm, c2_hbm,
              out_hbm, mask_hbm,
              e_v, p_v, c2_v, out_v, mask_v, s0, s1, s2):
        wid = lax.axis_index("s") * _NC + lax.axis_index("c")
        base = wid * ipw
        d0 = pltpu.async_copy(e_hbm.at[pl.ds(base * _H, ipw * _H)], e_v, s0)
        d1 = pltpu.async_copy(p_hbm.at[pl.ds(base, ipw)], p_v, s1)
        d2 = pltpu.async_copy(c2_hbm, c2_v, s2)
        d0.wait()
        d1.wait()
        d2.wait()

        iota = lax.iota(jnp.int32, _LANES)

        @plsc.parallel_loop(0, ipw, 1, unroll=8)
        def item(i):
            ivec = jnp.full((_LANES,), i, jnp.int32)
            pv = plsc.load_gather(p_v, [ivec])
            packed = plsc.load_gather(c2_v, [pv * _L + iota])
            hm_row = packed & jnp.int32(7)
            c2row = plsc.bitcast(packed & jnp.int32(-8), jnp.float32)
            e_row = plsc.load_gather(e_v, [ivec * _H + hm_row])
            pred = 1.0 / (1.0 + jnp.exp(-(e_row + c2row)))
            out_v[pl.ds(i * _L, _L)] = pred
            mask_v[pl.ds(i * _L, _L)] = jnp.where(
                c2row > -1e29, jnp.float32(1.0), jnp.float32(0.0))

        o0 = pltpu.async_copy(out_v, out_hbm.at[pl.ds(base * _L, ipw * _L)], s0)
        o1 = pltpu.async_copy(mask_v, mask_hbm.at[pl.ds(base * _L, ipw * _L)], s1)
        o0.wait()
        o1.wait()

    return route


def kernel(embedding, criteria, W, b, problem_indices):
    batch, d = embedding.shape
    crit_flat = criteria.reshape(_P * _L, d)
    e_scores, c2 = _scores_matmul(
        embedding, crit_flat, b.reshape(1, _H).astype(jnp.float32), W.T)

    route = _make_sc_route(batch)
    out_flat, mask_flat = route(
        e_scores.reshape(-1), problem_indices, c2.reshape(-1))
    return out_flat.reshape(batch, _L), mask_flat.reshape(batch, _L)


# speedup vs baseline: 1.1226x; 1.0018x over previous
import functools

import numpy as np
import jax
import jax.numpy as jnp
from jax import lax
from jax.experimental import pallas as pl
from jax.experimental.pallas import tpu as pltpu
from jax.experimental.pallas import tpu_sc as plsc

_CRITERIA_TO_HEAD = [
    [0, 1, 2, 3, 4, 5, 6, 7, 0, 1, 2, 3],
    [1, 2, 3, 4, 5, 6, 7, 0],
    [2, 3, 4, 5, 6, 7, 0, 1, 2, 3, 4, 5, 6, 7, 0, 1],
    [3, 4, 5, 6],
    [4, 5, 6, 7, 0, 1, 2, 3, 4, 5],
    [5, 6, 7, 0, 1, 2],
    [6, 7, 0, 1, 2, 3, 4, 5, 6, 7, 0, 1, 2, 3],
    [7, 0],
]
_L = 16
_P = len(_CRITERIA_TO_HEAD)
_H = 8

_LEN_NP = np.array([len(m) for m in _CRITERIA_TO_HEAD], dtype=np.int32)
_HM_NP = np.zeros((_P, _L), dtype=np.int32)
for _i, _m in enumerate(_CRITERIA_TO_HEAD):
    _HM_NP[_i, : len(_m)] = np.array(_m, dtype=np.int32)
_ONEHOT_NP = (_HM_NP.reshape(-1, 1) == np.arange(_H)[None, :]).astype(np.float32)
_NEGINF_NP = np.where(
    np.arange(_L)[None, :] < _LEN_NP[:, None], 0.0, -1e30
).astype(np.float32).reshape(-1)

_NC, _NS, _LANES = 2, 16, 16
_NW = _NC * _NS


def _mm_body(x_ref, cf_ref, b_ref, wt_ref, oh_ref, ninf_ref, hm_ref,
             e_ref, c2_ref):
    wt = wt_ref[...]
    e_ref[...] = jnp.dot(x_ref[...], wt, preferred_element_type=jnp.float32)

    @pl.when(pl.program_id(0) == 0)
    def _():
        cw = jnp.dot(cf_ref[...], wt, preferred_element_type=jnp.float32)
        onehot = oh_ref[...]
        bsel = jnp.sum(onehot * b_ref[...], axis=1)
        c2 = jnp.sum(cw * onehot, axis=1) + bsel + ninf_ref[0, :]
        ci = lax.bitcast_convert_type(c2.reshape(1, _P * _L), jnp.int32)
        c2_ref[...] = (ci & jnp.int32(-8)) | hm_ref[...]


def _scores_matmul(emb, crit_flat, b_row, wt):
    rows, d = emb.shape
    h = wt.shape[1]
    blk = 1024
    grid = rows // blk
    return pl.pallas_call(
        _mm_body,
        grid=(grid,),
        in_specs=[
            pl.BlockSpec((blk, d), lambda i: (i, 0)),
            pl.BlockSpec((_P * _L, d), lambda i: (0, 0)),
            pl.BlockSpec((1, _H), lambda i: (0, 0)),
            pl.BlockSpec((d, h), lambda i: (0, 0)),
            pl.BlockSpec((_P * _L, _H), lambda i: (0, 0)),
            pl.BlockSpec((1, _P * _L), lambda i: (0, 0)),
            pl.BlockSpec((1, _P * _L), lambda i: (0, 0)),
        ],
        out_specs=[
            pl.BlockSpec((blk, h), lambda i: (i, 0)),
            pl.BlockSpec((1, _P * _L), lambda i: (0, 0)),
        ],
        out_shape=[
            jax.ShapeDtypeStruct((rows, h), jnp.float32),
            jax.ShapeDtypeStruct((1, _P * _L), jnp.int32),
        ],
    )(emb, crit_flat, b_row, wt,
      jnp.asarray(_ONEHOT_NP), jnp.asarray(_NEGINF_NP).reshape(1, _P * _L),
      jnp.asarray(_HM_NP.reshape(1, _P * _L)))


def _make_sc_route(batch):
    ipw = batch // _NW
    mesh = plsc.VectorSubcoreMesh(
        core_axis_name="c", subcore_axis_name="s",
        num_cores=_NC, num_subcores=_NS,
    )

    @functools.partial(
        pl.kernel,
        mesh=mesh,
        compiler_params=pltpu.CompilerParams(
            needs_layout_passes=False,
            skip_device_barrier=True,
            disable_bounds_checks=True,
        ),
        out_type=(
            jax.ShapeDtypeStruct((batch * _L,), jnp.float32),
            jax.ShapeDtypeStruct((batch * _L,), jnp.float32),
        ),
        scratch_types=(
            pltpu.VMEM((ipw * _H,), jnp.float32),
            pltpu.VMEM((ipw,), jnp.int32),
            pltpu.VMEM((_P * _L,), jnp.int32),
            pltpu.VMEM((ipw * _L,), jnp.float32),
            pltpu.VMEM((ipw * _L,), jnp.float32),
            pltpu.SemaphoreType.DMA,
            pltpu.SemaphoreType.DMA,
            pltpu.SemaphoreType.DMA,
        ),
    )
    def route(e_hbm, p_hbm, c2_hbm,
              out_hbm, mask_hbm,
              e_v, p_v, c2_v, out_v, mask_v, s0, s1, s2):
        wid = lax.axis_index("s") * _NC + lax.axis_index("c")
        base = wid * ipw
        d0 = pltpu.async_copy(e_hbm.at[pl.ds(base * _H, ipw * _H)], e_v, s0)
        d1 = pltpu.async_copy(p_hbm.at[pl.ds(base, ipw)], p_v, s1)
        d2 = pltpu.async_copy(c2_hbm, c2_v, s2)
        d0.wait()
        d1.wait()
        d2.wait()

        iota = lax.iota(jnp.int32, _LANES)

        @plsc.parallel_loop(0, ipw, 1, unroll=16)
        def item(i):
            ivec = jnp.full((_LANES,), i, jnp.int32)
            pv = plsc.load_gather(p_v, [ivec])
            packed = plsc.load_gather(c2_v, [pv * _L + iota])
            hm_row = packed & jnp.int32(7)
            c2row = plsc.bitcast(packed & jnp.int32(-8), jnp.float32)
            e_row = plsc.load_gather(e_v, [ivec * _H + hm_row])
            pred = 1.0 / (1.0 + jnp.exp(-(e_row + c2row)))
            out_v[pl.ds(i * _L, _L)] = pred
            mask_v[pl.ds(i * _L, _L)] = jnp.where(
                c2row > -1e29, jnp.float32(1.0), jnp.float32(0.0))

        o0 = pltpu.async_copy(out_v, out_hbm.at[pl.ds(base * _L, ipw * _L)], s0)
        o1 = pltpu.async_copy(mask_v, mask_hbm.at[pl.ds(base * _L, ipw * _L)], s1)
        o0.wait()
        o1.wait()

    return route


def kernel(embedding, criteria, W, b, problem_indices):
    batch, d = embedding.shape
    crit_flat = criteria.reshape(_P * _L, d)
    e_scores, c2 = _scores_matmul(
        embedding, crit_flat, b.reshape(1, _H).astype(jnp.float32), W.T)

    route = _make_sc_route(batch)
    out_flat, mask_flat = route(
        e_scores.reshape(-1), problem_indices, c2.reshape(-1))
    return out_flat.reshape(batch, _L), mask_flat.reshape(batch, _L)
